# baseline (speedup 1.0000x reference)
import functools

import jax
import jax.numpy as jnp
from jax import lax
from jax.experimental import pallas as pl
from jax.experimental.pallas import tpu as pltpu
from jax.experimental.pallas import tpu_sc as plsc

NUM_FIELDS = 26
VOCAB = 100000
EMBED = 32
BATCH = 16384

NC = 2
NS = 16
NW = NC * NS
L = 16

NPAIR = NUM_FIELDS * EMBED
PPW = NPAIR // NW
BCHUNK = 2048
NCB = BATCH // BCHUNK
GU = 16


@functools.partial(
    pl.kernel,
    mesh=plsc.VectorSubcoreMesh(core_axis_name="c", subcore_axis_name="s"),
    out_type=jax.ShapeDtypeStruct((NPAIR, BATCH), jnp.float32),
    scratch_types=[
        pltpu.VMEM((VOCAB,), jnp.float32),
        pltpu.VMEM((BATCH,), jnp.int32),
        pltpu.VMEM((BCHUNK,), jnp.float32),
        pltpu.VMEM((BCHUNK,), jnp.float32),
        pltpu.SemaphoreType.DMA,
        pltpu.SemaphoreType.DMA,
        pltpu.SemaphoreType.DMA,
        pltpu.SemaphoreType.DMA,
    ],
    compiler_params=pltpu.CompilerParams(
        use_tc_tiling_on_sc=True, needs_layout_passes=False
    ),
)
def _embed_gather(x_hbm, tab_hbm, out_hbm, slice_v, idx_v, row0, row1,
                  ssem, isem, osem0, osem1):
    rows = (row0, row1)
    osems = (osem0, osem1)

    wid = lax.axis_index("c") * NS + lax.axis_index("s")
    p0 = wid * PPW

    def out_start(p, cb, k):
        pltpu.async_copy(
            rows[k], out_hbm.at[p, pl.ds(cb * BCHUNK, BCHUNK)], osems[k]
        )

    def out_wait(k):
        pltpu.make_async_copy(
            rows[k], out_hbm.at[0, pl.ds(0, BCHUNK)], osems[k]
        ).wait()

    def gather_chunk(cb, k):
        def body(u, c):
            base = u * (L * GU)
            for g in range(GU):
                s = base + g * L
                vi = idx_v[pl.ds(cb * BCHUNK + s, L)]
                rows[k][pl.ds(s, L)] = plsc.load_gather(slice_v, [vi])
            return c

        lax.fori_loop(0, BCHUNK // (L * GU), body, 0)

    def pair_body(i, carry):
        p = p0 + i
        f = p // EMBED
        e = p % EMBED
        pltpu.async_copy(tab_hbm.at[f, e], slice_v, ssem)

        @pl.when(jnp.logical_or(i == 0, e == 0))
        def _():
            pltpu.async_copy(x_hbm.at[f], idx_v, isem)
            pltpu.make_async_copy(x_hbm.at[0], idx_v, isem).wait()

        pltpu.make_async_copy(tab_hbm.at[0, 0], slice_v, ssem).wait()
        for cb in range(NCB):
            k = cb % 2
            if cb >= 2:
                out_wait(k)
            else:
                @pl.when(i > 0)
                def _():
                    out_wait(k)
            gather_chunk(cb, k)
            out_start(p, cb, k)
        return carry

    lax.fori_loop(0, PPW, pair_body, 0)
    out_wait(0)
    out_wait(1)


def kernel(input_x, tables):
    x_t = jnp.transpose(input_x, (1, 0)).astype(jnp.int32)
    tab_t = jnp.transpose(tables, (0, 2, 1))
    out_t = _embed_gather(x_t, tab_t)
    return jnp.transpose(out_t, (1, 0))
